# baseline (speedup 1.0000x reference)
import functools

import jax
import jax.numpy as jnp
from jax import lax
from jax.experimental import pallas as pl
from jax.experimental.pallas import tpu as pltpu
from jax.experimental.pallas import tpu_sc as plsc

_D = 32
_NC = 2
_NS = 16
_NW = _NC * _NS

_HIST = 50
_BATCH = 4096
_VOCAB = 1000000

_CHUNK = 512
_CPH = _BATCH // _CHUNK
_NU = _HIST * _CPH
_STEPS = -(-_NU // _NW)

_NSLAB = _VOCAB // 128
_SROWS = _VOCAB // 4

_MAGIC = 305419896


def _make_phys_kernel():
    mesh = plsc.VectorSubcoreMesh(core_axis_name="c", subcore_axis_name="s")

    @functools.partial(
        pl.kernel,
        mesh=mesh,
        out_type=(
            jax.ShapeDtypeStruct((_HIST, _D, _BATCH), jnp.float32),
            jax.ShapeDtypeStruct((_SROWS, 128), jnp.float32),
            jax.ShapeDtypeStruct((16, 128), jnp.int32),
        ),
        scratch_types=[
            pltpu.VMEM((_D, 128), jnp.float32),
            pltpu.VMEM((_D, 128), jnp.float32),
            pltpu.VMEM((8, _CHUNK), jnp.int32),
            pltpu.VMEM((4, 128), jnp.int32),
            pltpu.VMEM((_CHUNK,), jnp.int32),
            pltpu.VMEM((_CHUNK, 128), jnp.float32),
            pltpu.VMEM((_D, _CHUNK), jnp.float32),
            pltpu.VMEM((8, 128), jnp.int32),
            pltpu.VMEM((8, 128), jnp.int32),
            pltpu.SemaphoreType.DMA,
            pltpu.SemaphoreType.DMA,
            pltpu.SemaphoreType.DMA,
        ],
        compiler_params=pltpu.CompilerParams(
            use_tc_tiling_on_sc=True, needs_layout_passes=False
        ),
    )
    def phys_kernel(idxT, tabT, tailS, outP, s_hbm, flag, tbuf, sbuf, idxbuf,
                    glist, mrow, gbuf, obuf, magic_v, fbuf, dsem, gsem, wsem):
        cid = lax.axis_index("c")
        sid = lax.axis_index("s")
        wid = sid * _NC + cid
        iota = lax.iota(jnp.int32, 16)

        def repack_slab(tile_lo, s_lo, nrows):
            for j0 in (0, 16):
                i0 = iota + j0
                for s in range(nrows):
                    for a in range(4):
                        i1 = jnp.full((16,), tile_lo + 4 * s + a, jnp.int32)
                        v = plsc.load_gather(tbuf, [i0, i1])
                        sbuf[s, pl.ds(32 * a + j0, 16)] = v
            pltpu.sync_copy(
                sbuf.at[pl.ds(0, nrows), :], s_hbm.at[pl.ds(s_lo, nrows), :]
            )

        t_start = (wid * _NSLAB) // _NW
        t_end = ((wid + 1) * _NSLAB) // _NW

        def slab_body(t, _):
            off = pl.multiple_of(t * 128, 128)
            pltpu.sync_copy(tabT.at[:, pl.ds(off, 128)], tbuf)
            repack_slab(0, pl.multiple_of(t * 32, 8), 32)
            return 0

        lax.fori_loop(t_start, t_end, slab_body, 0)

        @pl.when(wid == _NW - 1)
        def _():
            pltpu.sync_copy(tailS, tbuf.at[pl.ds(0, 16), :])
            pltpu.sync_copy(
                tbuf.at[pl.ds(0, 16), :], s_hbm.at[pl.ds(_NSLAB * 32, 16), :]
            )

        magic_v[0, pl.ds(0, 16)] = jnp.full((16,), _MAGIC, jnp.int32)
        plsc.subcore_barrier()

        @pl.when(sid == 0)
        def _():
            pltpu.sync_copy(magic_v, flag.at[pl.ds(pl.multiple_of(8 * cid, 8), 8), :])

        def poll_cond(not_done):
            return not_done

        def poll_body(_):
            pltpu.sync_copy(flag.at[pl.ds(pl.multiple_of(8 * (1 - cid), 8), 8), :], fbuf)
            got = fbuf[0, pl.ds(0, 16)]
            ok = jax.lax.reduce_and(got == _MAGIC, axes=(0,))
            return jnp.logical_not(ok)

        lax.while_loop(poll_cond, poll_body, jnp.bool_(True))

        def do_unit(u):
            h = u // _CPH
            c0 = pl.multiple_of((u % _CPH) * _CHUNK, 128)
            pltpu.sync_copy(
                idxT.at[pl.ds(pl.multiple_of((h // 8) * 8, 8), 8), pl.ds(c0, _CHUNK)], idxbuf
            )
            hr = h % 8
            for k in range(_CHUNK // 16):
                iv = idxbuf[hr, pl.ds(16 * k, 16)]
                glist[k // 8, pl.ds(16 * (k % 8), 16)] = jax.lax.shift_right_logical(iv, 2)
                mrow[pl.ds(16 * k, 16)] = jax.lax.shift_left(iv & 3, 5)
            for q in range(4):
                pltpu.async_copy(
                    s_hbm.at[glist.at[q]], gbuf.at[pl.ds(128 * q, 128), :], gsem
                )
            pltpu.make_async_copy(
                s_hbm.at[pl.ds(0, _CHUNK)], gbuf, gsem
            ).wait()
            for j in range(_D):
                def chunk_body(c, _):
                    mv = mrow[pl.ds(16 * c, 16)] + j
                    b0 = 16 * c + iota
                    v = plsc.load_gather(gbuf, [b0, mv])
                    obuf[j, pl.ds(16 * c, 16)] = v
                    return 0

                lax.fori_loop(0, _CHUNK // 16, chunk_body, 0)
            pltpu.sync_copy(obuf, outP.at[h, :, pl.ds(c0, _CHUNK)])

        def step_body(step, _):
            u = step * _NW + wid

            @pl.when(u < _NU)
            def _():
                do_unit(u)

            return 0

        lax.fori_loop(0, _STEPS, step_body, 0)

    return phys_kernel


def kernel(indices, table):
    idxT = indices.T.astype(jnp.int32)
    tabT = table.T
    tailS = table[_NSLAB * 128:].reshape(16, 128)
    outP, _, _ = _make_phys_kernel()(idxT, tabT, tailS)
    return outP.transpose(2, 0, 1)


# speedup vs baseline: 1.7431x; 1.7431x over previous
import functools

import jax
import jax.numpy as jnp
from jax import lax
from jax.experimental import pallas as pl
from jax.experimental.pallas import tpu as pltpu
from jax.experimental.pallas import tpu_sc as plsc

_D = 32
_NC = 2
_NS = 16
_NW = _NC * _NS

_HIST = 50
_BATCH = 4096
_VOCAB = 1000000

_CHUNK = 256
_CPH = _BATCH // _CHUNK
_NU = _HIST * _CPH
_STEPS = _NU // _NW

_NSLAB = _VOCAB // 128
_SROWS = _VOCAB // 4
_BS = 1
_NB = _NSLAB // _BS

_MAGIC = 305419896


def _make_phys_kernel():
    mesh = plsc.VectorSubcoreMesh(core_axis_name="c", subcore_axis_name="s")

    @functools.partial(
        pl.kernel,
        mesh=mesh,
        out_type=(
            jax.ShapeDtypeStruct((_HIST, _D, _BATCH), jnp.float32),
            jax.ShapeDtypeStruct((_SROWS, 128), jnp.float32),
            jax.ShapeDtypeStruct((16, 128), jnp.int32),
        ),
        scratch_types=[
            [pltpu.VMEM((_D, 128 * _BS), jnp.float32) for _ in range(2)],
            [pltpu.VMEM((32 * _BS, 128), jnp.float32) for _ in range(2)],
            pltpu.VMEM((_D, 128), jnp.float32),
            [pltpu.VMEM((8, _CHUNK), jnp.int32) for _ in range(2)],
            [pltpu.VMEM((2, 128), jnp.int32) for _ in range(2)],
            [pltpu.VMEM((_CHUNK,), jnp.int32) for _ in range(2)],
            [pltpu.VMEM((_CHUNK, 128), jnp.float32) for _ in range(2)],
            [pltpu.VMEM((_D, _CHUNK), jnp.float32) for _ in range(2)],
            pltpu.VMEM((8, 128), jnp.int32),
            pltpu.VMEM((8, 128), jnp.int32),
            [pltpu.SemaphoreType.DMA for _ in range(2)],
            [pltpu.SemaphoreType.DMA for _ in range(2)],
            [pltpu.SemaphoreType.DMA for _ in range(2)],
            [pltpu.SemaphoreType.DMA for _ in range(2)],
            pltpu.SemaphoreType.DMA,
        ],
        compiler_params=pltpu.CompilerParams(
            use_tc_tiling_on_sc=True, needs_layout_passes=False
        ),
    )
    def phys_kernel(idxT, tabT, tailS, outP, s_hbm, flag,
                    tbufs, sbufs, skew, idxbufs, glists, mrows, gbufs, obufs,
                    magic_v, fbuf, isems, osems, gsems, wsems, dsem):
        cid = lax.axis_index("c")
        sid = lax.axis_index("s")
        wid = sid * _NC + cid
        iota = lax.iota(jnp.int32, 16)

        b_start = (wid * _NB) // _NW
        b_end = ((wid + 1) * _NB) // _NW

        def in_slice(b):
            off = pl.multiple_of(b * (128 * _BS), 128)
            return tabT.at[:, pl.ds(off, 128 * _BS)]

        def out_slice(b):
            off = pl.multiple_of(b * (32 * _BS), 8)
            return s_hbm.at[pl.ds(off, 32 * _BS), :]

        def fire_in(b, p):
            pltpu.async_copy(in_slice(b), tbufs[p], isems[p])

        def transpose_batch(p):
            tb, sb = tbufs[p], sbufs[p]
            for q in range(_BS):
                def skew_body(j, _):
                    jrow = jnp.full((16,), j, jnp.int32)
                    rot = (iota + j) & 15
                    for kk in range(8):
                        v = tb[j, pl.ds(128 * q + 16 * kk, 16)]
                        plsc.store_scatter(skew, [jrow, 16 * kk + rot], v)
                    return 0

                lax.fori_loop(0, _D, skew_body, 0)

                def col_body(c, _):
                    s = jax.lax.shift_right_logical(c, 2)
                    a = c & 3
                    cbase = c & ~15
                    for j0 in (0, 16):
                        lane = cbase + ((c + j0 + iota) & 15)
                        v = plsc.load_gather(skew, [iota + j0, lane])
                        sb[32 * q + s, pl.ds(a * 32 + j0, 16)] = v
                    return 0

                lax.fori_loop(0, 128, col_body, 0)

        def stage1():
            nloc = b_end - b_start

            @pl.when(nloc > 0)
            def _():
                fire_in(b_start, 0)

            def group_body(g, _):
                for p in range(2):
                    b = b_start + 2 * g + p

                    @pl.when(b < b_end)
                    def _():
                        nxt = b + 1
                        pltpu.make_async_copy(
                            in_slice(b), tbufs[p], isems[p]
                        ).wait()

                        @pl.when(nxt < b_end)
                        def _():
                            fire_in(nxt, 1 - p)

                        @pl.when(b - 2 >= b_start)
                        def _():
                            pltpu.make_async_copy(
                                sbufs[p], out_slice(b - 2), osems[p]
                            ).wait()

                        transpose_batch(p)
                        pltpu.async_copy(sbufs[p], out_slice(b), osems[p])

                return 0

            lax.fori_loop(0, (nloc + 1) // 2, group_body, 0)
            for p in range(2):
                @pl.when(nloc >= p + 1)
                def _():
                    pltpu.make_async_copy(
                        sbufs[p], out_slice(b_start), osems[p]
                    ).wait()

        stage1()

        @pl.when(wid == _NW - 1)
        def _():
            pltpu.sync_copy(tailS, sbufs[0].at[pl.ds(0, 16), :])
            pltpu.sync_copy(
                sbufs[0].at[pl.ds(0, 16), :],
                s_hbm.at[pl.ds(_NSLAB * 32, 16), :],
            )

        magic_v[0, pl.ds(0, 16)] = jnp.full((16,), _MAGIC, jnp.int32)
        plsc.subcore_barrier()

        @pl.when(sid == 0)
        def _():
            pltpu.sync_copy(
                magic_v, flag.at[pl.ds(pl.multiple_of(8 * cid, 8), 8), :]
            )

        def poll_body(_):
            pltpu.sync_copy(
                flag.at[pl.ds(pl.multiple_of(8 * (1 - cid), 8), 8), :], fbuf
            )
            got = fbuf[0, pl.ds(0, 16)]
            ok = jax.lax.reduce_and(got == _MAGIC, axes=(0,))
            return jnp.logical_not(ok)

        lax.while_loop(lambda nd: nd, poll_body, jnp.bool_(True))

        def unit_of(step):
            u = step * _NW + wid
            h = u // _CPH
            c0 = pl.multiple_of((u % _CPH) * _CHUNK, 128)
            return h, c0

        def prep_and_fire(step, p):
            h, c0 = unit_of(step)
            pltpu.sync_copy(
                idxT.at[
                    pl.ds(pl.multiple_of((h // 8) * 8, 8), 8), pl.ds(c0, _CHUNK)
                ],
                idxbufs[p],
            )
            hr = h % 8
            for k in range(_CHUNK // 16):
                iv = idxbufs[p][hr, pl.ds(16 * k, 16)]
                glists[p][k // 8, pl.ds(16 * (k % 8), 16)] = (
                    jax.lax.shift_right_logical(iv, 2)
                )
                mrows[p][pl.ds(16 * k, 16)] = jax.lax.shift_left(iv & 3, 5)
            for q in range(_CHUNK // 128):
                pltpu.async_copy(
                    s_hbm.at[glists[p].at[q]],
                    gbufs[p].at[pl.ds(128 * q, 128), :],
                    gsems[p],
                )

        def extract(step, p):
            gb, ob, mr = gbufs[p], obufs[p], mrows[p]
            for j in range(_D):
                def chunk_body(c, _):
                    mv = mr[pl.ds(16 * c, 16)] + j
                    b0 = 16 * c + iota
                    v = plsc.load_gather(gb, [b0, mv])
                    ob[j, pl.ds(16 * c, 16)] = v
                    return 0

                lax.fori_loop(0, _CHUNK // 16, chunk_body, 0)

        def write_out(step, p):
            h, c0 = unit_of(step)
            pltpu.async_copy(obufs[p], outP.at[h, :, pl.ds(c0, _CHUNK)], wsems[p])

        def wait_write(step, p):
            h, c0 = unit_of(step)
            pltpu.make_async_copy(
                obufs[p], outP.at[h, :, pl.ds(c0, _CHUNK)], wsems[p]
            ).wait()

        def drain_gather(p):
            pltpu.make_async_copy(
                s_hbm.at[pl.ds(0, _CHUNK)], gbufs[p], gsems[p]
            ).wait()

        prep_and_fire(0, 0)

        def s2_group(g, _):
            for p in range(2):
                step = 2 * g + p

                @pl.when(step < _STEPS)
                def _():
                    drain_gather(p)

                    @pl.when(step + 1 < _STEPS)
                    def _():
                        prep_and_fire(step + 1, 1 - p)

                    @pl.when(step - 2 >= 0)
                    def _():
                        wait_write(step - 2, p)

                    extract(step, p)
                    write_out(step, p)

            return 0

        lax.fori_loop(0, (_STEPS + 1) // 2, s2_group, 0)
        for step in (_STEPS - 2, _STEPS - 1):
            wait_write(step, step % 2)

    return phys_kernel


def kernel(indices, table):
    idxT = indices.T.astype(jnp.int32)
    tabT = table.T
    tailS = table[_NSLAB * 128:].reshape(16, 128)
    outP, _, _ = _make_phys_kernel()(idxT, tabT, tailS)
    return outP.transpose(2, 0, 1)


# speedup vs baseline: 2.1637x; 1.2413x over previous
import functools

import jax
import jax.numpy as jnp
from jax import lax
from jax.experimental import pallas as pl
from jax.experimental.pallas import tpu as pltpu
from jax.experimental.pallas import tpu_sc as plsc

_D = 32
_NC = 2
_NS = 16
_NW = _NC * _NS

_HIST = 50
_BATCH = 4096
_VOCAB = 1000000

_CHUNK = 256
_CPH = _BATCH // _CHUNK
_NU = _HIST * _CPH
_STEPS = _NU // _NW

_NSLAB = _VOCAB // 128
_SROWS = _VOCAB // 4
_BS = 1
_NB = _NSLAB // _BS

_MAGIC = 305419896


def _make_phys_kernel():
    mesh = plsc.VectorSubcoreMesh(core_axis_name="c", subcore_axis_name="s")

    @functools.partial(
        pl.kernel,
        mesh=mesh,
        out_type=(
            jax.ShapeDtypeStruct((_HIST, _D, _BATCH), jnp.float32),
            jax.ShapeDtypeStruct((_SROWS, 128), jnp.float32),
            jax.ShapeDtypeStruct((16, 128), jnp.int32),
        ),
        scratch_types=[
            [pltpu.VMEM((_D, 128 * _BS), jnp.float32) for _ in range(2)],
            [pltpu.VMEM((32 * _BS, 128), jnp.float32) for _ in range(2)],
            pltpu.VMEM((_D, 128), jnp.float32),
            [pltpu.VMEM((8, _CHUNK), jnp.int32) for _ in range(2)],
            [pltpu.VMEM((2, 128), jnp.int32) for _ in range(2)],
            [pltpu.VMEM((_CHUNK,), jnp.int32) for _ in range(2)],
            [pltpu.VMEM((_CHUNK, 128), jnp.float32) for _ in range(2)],
            [[pltpu.VMEM((_D, 128), jnp.float32) for _ in range(2)]
             for _ in range(2)],
            pltpu.VMEM((8, 128), jnp.int32),
            pltpu.VMEM((8, 128), jnp.int32),
            [pltpu.SemaphoreType.DMA for _ in range(2)],
            [pltpu.SemaphoreType.DMA for _ in range(2)],
            [pltpu.SemaphoreType.DMA for _ in range(2)],
            [pltpu.SemaphoreType.DMA for _ in range(2)],
            pltpu.SemaphoreType.DMA,
        ],
        compiler_params=pltpu.CompilerParams(
            use_tc_tiling_on_sc=True, needs_layout_passes=False
        ),
    )
    def phys_kernel(idxT, tabT, tailS, outP, s_hbm, flag,
                    tbufs, sbufs, skew, idxbufs, glists, mrows, gbufs, obufs,
                    magic_v, fbuf, isems, osems, gsems, wsems, dsem):
        cid = lax.axis_index("c")
        sid = lax.axis_index("s")
        wid = sid * _NC + cid
        iota = lax.iota(jnp.int32, 16)

        b_start = (wid * _NB) // _NW
        b_end = ((wid + 1) * _NB) // _NW

        def in_slice(b):
            off = pl.multiple_of(b * (128 * _BS), 128)
            return tabT.at[:, pl.ds(off, 128 * _BS)]

        def out_slice(b):
            off = pl.multiple_of(b * (32 * _BS), 8)
            return s_hbm.at[pl.ds(off, 32 * _BS), :]

        def fire_in(b, p):
            pltpu.async_copy(in_slice(b), tbufs[p], isems[p])

        def transpose_batch(p):
            tb, sb = tbufs[p], sbufs[p]
            for q in range(_BS):
                def skew_body(jg, _):
                    for jj in range(4):
                        j = 4 * jg + jj
                        jrow = jnp.full((16,), j, jnp.int32)
                        rot = (iota + j) & 15
                        for kk in range(8):
                            v = tb[j, pl.ds(128 * q + 16 * kk, 16)]
                            plsc.store_scatter(skew, [jrow, 16 * kk + rot], v)
                    return 0

                lax.fori_loop(0, _D // 4, skew_body, 0)

                def col_body(s, _):
                    for a in range(4):
                        c = 4 * s + a
                        cbase = c & ~15
                        for j0 in (0, 16):
                            lane = cbase + ((c + j0 + iota) & 15)
                            v = plsc.load_gather(skew, [iota + j0, lane])
                            sb[32 * q + s, pl.ds(a * 32 + j0, 16)] = v
                    return 0

                lax.fori_loop(0, 32, col_body, 0)

        def stage1():
            nloc = b_end - b_start

            @pl.when(nloc > 0)
            def _():
                fire_in(b_start, 0)

            def group_body(g, _):
                for p in range(2):
                    b = b_start + 2 * g + p

                    @pl.when(b < b_end)
                    def _():
                        nxt = b + 1
                        pltpu.make_async_copy(
                            in_slice(b), tbufs[p], isems[p]
                        ).wait()

                        @pl.when(nxt < b_end)
                        def _():
                            fire_in(nxt, 1 - p)

                        @pl.when(b - 2 >= b_start)
                        def _():
                            pltpu.make_async_copy(
                                sbufs[p], out_slice(b - 2), osems[p]
                            ).wait()

                        transpose_batch(p)
                        pltpu.async_copy(sbufs[p], out_slice(b), osems[p])

                return 0

            lax.fori_loop(0, (nloc + 1) // 2, group_body, 0)
            for p in range(2):
                @pl.when(nloc >= p + 1)
                def _():
                    pltpu.make_async_copy(
                        sbufs[p], out_slice(b_start), osems[p]
                    ).wait()

        stage1()

        @pl.when(wid == _NW - 1)
        def _():
            pltpu.sync_copy(tailS, sbufs[0].at[pl.ds(0, 16), :])
            pltpu.sync_copy(
                sbufs[0].at[pl.ds(0, 16), :],
                s_hbm.at[pl.ds(_NSLAB * 32, 16), :],
            )

        magic_v[0, pl.ds(0, 16)] = jnp.full((16,), _MAGIC, jnp.int32)
        plsc.subcore_barrier()

        @pl.when(sid == 0)
        def _():
            pltpu.sync_copy(
                magic_v, flag.at[pl.ds(pl.multiple_of(8 * cid, 8), 8), :]
            )

        def poll_body(_):
            pltpu.sync_copy(
                flag.at[pl.ds(pl.multiple_of(8 * (1 - cid), 8), 8), :], fbuf
            )
            got = fbuf[0, pl.ds(0, 16)]
            ok = jax.lax.reduce_and(got == _MAGIC, axes=(0,))
            return jnp.logical_not(ok)

        lax.while_loop(lambda nd: nd, poll_body, jnp.bool_(True))

        def unit_of(step):
            u = step * _NW + wid
            h = u // _CPH
            c0 = pl.multiple_of((u % _CPH) * _CHUNK, 128)
            return h, c0

        def prep_and_fire(step, p):
            h, c0 = unit_of(step)
            pltpu.sync_copy(
                idxT.at[
                    pl.ds(pl.multiple_of((h // 8) * 8, 8), 8), pl.ds(c0, _CHUNK)
                ],
                idxbufs[p],
            )
            hr = h % 8
            for k in range(_CHUNK // 16):
                iv = idxbufs[p][hr, pl.ds(16 * k, 16)]
                glists[p][k // 8, pl.ds(16 * (k % 8), 16)] = (
                    jax.lax.shift_right_logical(iv, 2)
                )
                mrows[p][pl.ds(16 * k, 16)] = jax.lax.shift_left(iv & 3, 5)
            for q in range(_CHUNK // 128):
                pltpu.async_copy(
                    s_hbm.at[glists[p].at[q]],
                    gbufs[p].at[pl.ds(128 * q, 128), :],
                    gsems[p],
                )

        def extract(step, p):
            gb, mr = gbufs[p], mrows[p]
            for half in range(2):
                ob = obufs[p][half]

                def c_body(c, _):
                    lbase = 16 * c

                    def d_body(d, _):
                        lane = lbase + ((iota + d) & 15)
                        b_l = 128 * half + lane
                        mvp = plsc.load_gather(mr, [b_l])
                        for j0 in (0, 16):
                            v = plsc.load_gather(gb, [b_l, mvp + (iota + j0)])
                            plsc.store_scatter(ob, [iota + j0, lane], v)
                        return 0

                    lax.fori_loop(0, 16, d_body, 0)
                    return 0

                lax.fori_loop(0, 8, c_body, 0)

        def write_out(step, p):
            h, c0 = unit_of(step)
            for half in range(2):
                pltpu.async_copy(
                    obufs[p][half],
                    outP.at[h, :, pl.ds(c0 + 128 * half, 128)],
                    wsems[p],
                )

        def wait_write(step, p):
            h, c0 = unit_of(step)
            for half in range(2):
                pltpu.make_async_copy(
                    obufs[p][half],
                    outP.at[h, :, pl.ds(c0 + 128 * half, 128)],
                    wsems[p],
                ).wait()

        def drain_gather(p):
            pltpu.make_async_copy(
                s_hbm.at[pl.ds(0, _CHUNK)], gbufs[p], gsems[p]
            ).wait()

        prep_and_fire(0, 0)

        def s2_group(g, _):
            for p in range(2):
                step = 2 * g + p

                @pl.when(step < _STEPS)
                def _():
                    drain_gather(p)

                    @pl.when(step + 1 < _STEPS)
                    def _():
                        prep_and_fire(step + 1, 1 - p)

                    @pl.when(step - 2 >= 0)
                    def _():
                        wait_write(step - 2, p)

                    extract(step, p)
                    write_out(step, p)

            return 0

        lax.fori_loop(0, (_STEPS + 1) // 2, s2_group, 0)
        for step in (_STEPS - 2, _STEPS - 1):
            wait_write(step, step % 2)

    return phys_kernel


def kernel(indices, table):
    idxT = indices.T.astype(jnp.int32)
    tabT = table.T
    tailS = table[_NSLAB * 128:].reshape(16, 128)
    outP, _, _ = _make_phys_kernel()(idxT, tabT, tailS)
    return outP.transpose(2, 0, 1)


# speedup vs baseline: 3.8035x; 1.7579x over previous
import functools

import jax
import jax.numpy as jnp
from jax import lax
from jax.experimental import pallas as pl
from jax.experimental.pallas import tpu as pltpu
from jax.experimental.pallas import tpu_sc as plsc

_D = 32
_NC = 2
_NS = 16
_NW = _NC * _NS

_HIST = 50
_BATCH = 4096
_VOCAB = 1000000

_CHUNK = 256
_CPH = _BATCH // _CHUNK
_NU = _HIST * _CPH
_STEPS = _NU // _NW

_NSLAB = _VOCAB // 128
_SROWS = _VOCAB // 4
_BS = 1
_NB = _NSLAB // _BS

_MAGIC = 305419896


def _make_phys_kernel():
    mesh = plsc.VectorSubcoreMesh(core_axis_name="c", subcore_axis_name="s")

    @functools.partial(
        pl.kernel,
        mesh=mesh,
        out_type=(
            jax.ShapeDtypeStruct((_HIST, _D, _BATCH), jnp.float32),
            jax.ShapeDtypeStruct((_SROWS, 128), jnp.float32),
            jax.ShapeDtypeStruct((16, 128), jnp.int32),
        ),
        scratch_types=[
            [pltpu.VMEM((_D, 128 * _BS), jnp.float32) for _ in range(2)],
            [pltpu.VMEM((32 * _BS, 128), jnp.float32) for _ in range(2)],
            pltpu.VMEM((_D, 128), jnp.float32),
            [pltpu.VMEM((8, _CHUNK), jnp.int32) for _ in range(2)],
            [pltpu.VMEM((2, 128), jnp.int32) for _ in range(2)],
            [pltpu.VMEM((_CHUNK,), jnp.int32) for _ in range(2)],
            [pltpu.VMEM((_CHUNK, 128), jnp.float32) for _ in range(2)],
            [[pltpu.VMEM((_D, 128), jnp.float32) for _ in range(2)]
             for _ in range(2)],
            pltpu.VMEM((8, 128), jnp.int32),
            pltpu.VMEM((8, 128), jnp.int32),
            [pltpu.SemaphoreType.DMA for _ in range(2)],
            [pltpu.SemaphoreType.DMA for _ in range(2)],
            [pltpu.SemaphoreType.DMA for _ in range(2)],
            [pltpu.SemaphoreType.DMA for _ in range(2)],
            pltpu.SemaphoreType.DMA,
        ],
        compiler_params=pltpu.CompilerParams(
            use_tc_tiling_on_sc=True, needs_layout_passes=False
        ),
    )
    def phys_kernel(idxT, tabT, tailS, outP, s_hbm, flag,
                    tbufs, sbufs, skew, idxbufs, glists, mrows, gbufs, obufs,
                    magic_v, fbuf, isems, osems, gsems, wsems, dsem):
        cid = lax.axis_index("c")
        sid = lax.axis_index("s")
        wid = sid * _NC + cid
        iota = lax.iota(jnp.int32, 16)

        b_start = (wid * _NB) // _NW
        b_end = ((wid + 1) * _NB) // _NW

        def in_slice(b):
            off = pl.multiple_of(b * (128 * _BS), 128)
            return tabT.at[:, pl.ds(off, 128 * _BS)]

        def out_slice(b):
            off = pl.multiple_of(b * (32 * _BS), 8)
            return s_hbm.at[pl.ds(off, 32 * _BS), :]

        def fire_in(b, p):
            pltpu.async_copy(in_slice(b), tbufs[p], isems[p])

        def transpose_batch(p):
            tb, sb = tbufs[p], sbufs[p]
            for q in range(_BS):
                def skew_body(jg, _):
                    loads = []
                    for jj in range(4):
                        j = 4 * jg + jj
                        jrow = jnp.full((16,), j, jnp.int32)
                        rot = (iota + j) & 15
                        for kk in range(8):
                            v = tb[j, pl.ds(128 * q + 16 * kk, 16)]
                            loads.append((jrow, 16 * kk + rot, v))
                    for jrow, lanes, v in loads:
                        plsc.store_scatter(skew, [jrow, lanes], v)
                    return 0

                lax.fori_loop(0, _D // 4, skew_body, 0)

                def col_body(s, _):
                    cbase = (4 * s) & ~15
                    gots = []
                    for a in range(4):
                        c = 4 * s + a
                        lane = cbase + ((c + iota) & 15)
                        for j0 in (0, 16):
                            v = plsc.load_gather(skew, [iota + j0, lane])
                            gots.append((a, j0, v))
                    for a, j0, v in gots:
                        sb[32 * q + s, pl.ds(a * 32 + j0, 16)] = v
                    return 0

                lax.fori_loop(0, 32, col_body, 0)

        def stage1():
            nloc = b_end - b_start

            @pl.when(nloc > 0)
            def _():
                fire_in(b_start, 0)

            def group_body(g, _):
                for p in range(2):
                    b = b_start + 2 * g + p

                    @pl.when(b < b_end)
                    def _():
                        nxt = b + 1
                        pltpu.make_async_copy(
                            in_slice(b), tbufs[p], isems[p]
                        ).wait()

                        @pl.when(nxt < b_end)
                        def _():
                            fire_in(nxt, 1 - p)

                        @pl.when(b - 2 >= b_start)
                        def _():
                            pltpu.make_async_copy(
                                sbufs[p], out_slice(b - 2), osems[p]
                            ).wait()

                        transpose_batch(p)
                        pltpu.async_copy(sbufs[p], out_slice(b), osems[p])

                return 0

            lax.fori_loop(0, (nloc + 1) // 2, group_body, 0)
            for p in range(2):
                @pl.when(nloc >= p + 1)
                def _():
                    pltpu.make_async_copy(
                        sbufs[p], out_slice(b_start), osems[p]
                    ).wait()

        stage1()

        @pl.when(wid == _NW - 1)
        def _():
            pltpu.sync_copy(tailS, sbufs[0].at[pl.ds(0, 16), :])
            pltpu.sync_copy(
                sbufs[0].at[pl.ds(0, 16), :],
                s_hbm.at[pl.ds(_NSLAB * 32, 16), :],
            )

        magic_v[0, pl.ds(0, 16)] = jnp.full((16,), _MAGIC, jnp.int32)
        plsc.subcore_barrier()

        @pl.when(sid == 0)
        def _():
            pltpu.sync_copy(
                magic_v, flag.at[pl.ds(pl.multiple_of(8 * cid, 8), 8), :]
            )

        def poll_body(_):
            pltpu.sync_copy(
                flag.at[pl.ds(pl.multiple_of(8 * (1 - cid), 8), 8), :], fbuf
            )
            got = fbuf[0, pl.ds(0, 16)]
            ok = jax.lax.reduce_and(got == _MAGIC, axes=(0,))
            return jnp.logical_not(ok)

        lax.while_loop(lambda nd: nd, poll_body, jnp.bool_(True))

        def unit_of(step):
            u = step * _NW + wid
            h = u // _CPH
            c0 = pl.multiple_of((u % _CPH) * _CHUNK, 128)
            return h, c0

        def prep_and_fire(step, p):
            h, c0 = unit_of(step)
            pltpu.sync_copy(
                idxT.at[
                    pl.ds(pl.multiple_of((h // 8) * 8, 8), 8), pl.ds(c0, _CHUNK)
                ],
                idxbufs[p],
            )
            hr = h % 8
            for k in range(_CHUNK // 16):
                iv = idxbufs[p][hr, pl.ds(16 * k, 16)]
                glists[p][k // 8, pl.ds(16 * (k % 8), 16)] = (
                    jax.lax.shift_right_logical(iv, 2)
                )
                mrows[p][pl.ds(16 * k, 16)] = jax.lax.shift_left(iv & 3, 5)
            for q in range(_CHUNK // 128):
                pltpu.async_copy(
                    s_hbm.at[glists[p].at[q]],
                    gbufs[p].at[pl.ds(128 * q, 128), :],
                    gsems[p],
                )

        def extract(step, p):
            gb, mr = gbufs[p], mrows[p]
            for half in range(2):
                ob = obufs[p][half]

                def c_body(c, _):
                    lbase = 16 * c

                    def d_body(d, _):
                        lane = lbase + ((iota + d) & 15)
                        b_l = 128 * half + lane
                        mvp = plsc.load_gather(mr, [b_l])
                        vs = [
                            plsc.load_gather(gb, [b_l, mvp + (iota + j0)])
                            for j0 in (0, 16)
                        ]
                        for j0, v in zip((0, 16), vs):
                            plsc.store_scatter(ob, [iota + j0, lane], v)
                        return 0

                    lax.fori_loop(0, 16, d_body, 0)
                    return 0

                lax.fori_loop(0, 8, c_body, 0)

        def write_out(step, p):
            h, c0 = unit_of(step)
            for half in range(2):
                pltpu.async_copy(
                    obufs[p][half],
                    outP.at[h, :, pl.ds(c0 + 128 * half, 128)],
                    wsems[p],
                )

        def wait_write(step, p):
            h, c0 = unit_of(step)
            for half in range(2):
                pltpu.make_async_copy(
                    obufs[p][half],
                    outP.at[h, :, pl.ds(c0 + 128 * half, 128)],
                    wsems[p],
                ).wait()

        def drain_gather(p):
            pltpu.make_async_copy(
                s_hbm.at[pl.ds(0, _CHUNK)], gbufs[p], gsems[p]
            ).wait()

        prep_and_fire(0, 0)

        def s2_group(g, _):
            for p in range(2):
                step = 2 * g + p

                @pl.when(step < _STEPS)
                def _():
                    drain_gather(p)

                    @pl.when(step + 1 < _STEPS)
                    def _():
                        prep_and_fire(step + 1, 1 - p)

                    @pl.when(step - 2 >= 0)
                    def _():
                        wait_write(step - 2, p)

                    extract(step, p)
                    write_out(step, p)

            return 0

        lax.fori_loop(0, (_STEPS + 1) // 2, s2_group, 0)
        for step in (_STEPS - 2, _STEPS - 1):
            wait_write(step, step % 2)

    return phys_kernel


def kernel(indices, table):
    idxT = indices.T.astype(jnp.int32)
    tabT = table.T
    tailS = table[_NSLAB * 128:].reshape(16, 128)
    outP, _, _ = _make_phys_kernel()(idxT, tabT, tailS)
    return outP.transpose(2, 0, 1)


# speedup vs baseline: 4.1474x; 1.0904x over previous
import functools

import jax
import jax.numpy as jnp
from jax import lax
from jax.experimental import pallas as pl
from jax.experimental.pallas import tpu as pltpu
from jax.experimental.pallas import tpu_sc as plsc

_D = 32
_NC = 2
_NS = 16
_NW = _NC * _NS

_HIST = 50
_BATCH = 4096
_VOCAB = 1000000

_CHUNK = 256
_CPH = _BATCH // _CHUNK
_NU = _HIST * _CPH
_STEPS = _NU // _NW

_NSLAB = _VOCAB // 128
_SROWS = _VOCAB // 4
_BS = 1
_NB = _NSLAB // _BS

_MAGIC = 305419896


def _make_phys_kernel():
    mesh = plsc.VectorSubcoreMesh(core_axis_name="c", subcore_axis_name="s")

    @functools.partial(
        pl.kernel,
        mesh=mesh,
        out_type=(
            jax.ShapeDtypeStruct((_HIST, _D, _BATCH), jnp.float32),
            jax.ShapeDtypeStruct((_SROWS, 128), jnp.float32),
            jax.ShapeDtypeStruct((16, 128), jnp.int32),
        ),
        scratch_types=[
            [pltpu.VMEM((_D, 128 * _BS), jnp.float32) for _ in range(2)],
            [pltpu.VMEM((32 * _BS, 128), jnp.float32) for _ in range(2)],
            pltpu.VMEM((_D, 128), jnp.float32),
            [pltpu.VMEM((8, _CHUNK), jnp.int32) for _ in range(2)],
            [pltpu.VMEM((2, 128), jnp.int32) for _ in range(2)],
            [pltpu.VMEM((_CHUNK,), jnp.int32) for _ in range(2)],
            [pltpu.VMEM((_CHUNK, 128), jnp.float32) for _ in range(2)],
            [[pltpu.VMEM((_D, 128), jnp.float32) for _ in range(2)]
             for _ in range(2)],
            pltpu.VMEM((8, 128), jnp.int32),
            pltpu.VMEM((8, 128), jnp.int32),
            [pltpu.SemaphoreType.DMA for _ in range(2)],
            [pltpu.SemaphoreType.DMA for _ in range(2)],
            [pltpu.SemaphoreType.DMA for _ in range(2)],
            [pltpu.SemaphoreType.DMA for _ in range(2)],
            pltpu.SemaphoreType.DMA,
        ],
        compiler_params=pltpu.CompilerParams(
            use_tc_tiling_on_sc=True, needs_layout_passes=False
        ),
    )
    def phys_kernel(idxT, tabT, tailS, outP, s_hbm, flag,
                    tbufs, sbufs, skew, idxbufs, glists, mrows, gbufs, obufs,
                    magic_v, fbuf, isems, osems, gsems, wsems, dsem):
        cid = lax.axis_index("c")
        sid = lax.axis_index("s")
        wid = sid * _NC + cid
        iota = lax.iota(jnp.int32, 16)

        b_start = (wid * _NB) // _NW
        b_end = ((wid + 1) * _NB) // _NW

        def in_slice(b):
            off = pl.multiple_of(b * (128 * _BS), 128)
            return tabT.at[:, pl.ds(off, 128 * _BS)]

        def out_slice(b):
            off = pl.multiple_of(b * (32 * _BS), 8)
            return s_hbm.at[pl.ds(off, 32 * _BS), :]

        def fire_in(b, p):
            pltpu.async_copy(in_slice(b), tbufs[p], isems[p])

        def transpose_batch(p):
            tb, sb = tbufs[p], sbufs[p]
            for q in range(_BS):
                def skew_body(jg, _):
                    loads = []
                    for jj in range(8):
                        j = 8 * jg + jj
                        jrow = jnp.full((16,), j, jnp.int32)
                        rot = (iota + j) & 15
                        for kk in range(8):
                            v = tb[j, pl.ds(128 * q + 16 * kk, 16)]
                            loads.append((jrow, 16 * kk + rot, v))
                    for jrow, lanes, v in loads:
                        plsc.store_scatter(skew, [jrow, lanes], v)
                    return 0

                lax.fori_loop(0, _D // 8, skew_body, 0)

                def col_body(sg, _):
                    gots = []
                    for ss in range(2):
                        s = 2 * sg + ss
                        cbase = (4 * s) & ~15
                        for a in range(4):
                            c = 4 * s + a
                            lane = cbase + ((c + iota) & 15)
                            for j0 in (0, 16):
                                v = plsc.load_gather(skew, [iota + j0, lane])
                                gots.append((s, a, j0, v))
                    for s, a, j0, v in gots:
                        sb[32 * q + s, pl.ds(a * 32 + j0, 16)] = v
                    return 0

                lax.fori_loop(0, 16, col_body, 0)

        def stage1():
            nloc = b_end - b_start

            @pl.when(nloc > 0)
            def _():
                fire_in(b_start, 0)

            def group_body(g, _):
                for p in range(2):
                    b = b_start + 2 * g + p

                    @pl.when(b < b_end)
                    def _():
                        nxt = b + 1
                        pltpu.make_async_copy(
                            in_slice(b), tbufs[p], isems[p]
                        ).wait()

                        @pl.when(nxt < b_end)
                        def _():
                            fire_in(nxt, 1 - p)

                        @pl.when(b - 2 >= b_start)
                        def _():
                            pltpu.make_async_copy(
                                sbufs[p], out_slice(b - 2), osems[p]
                            ).wait()

                        transpose_batch(p)
                        pltpu.async_copy(sbufs[p], out_slice(b), osems[p])

                return 0

            lax.fori_loop(0, (nloc + 1) // 2, group_body, 0)
            for p in range(2):
                @pl.when(nloc >= p + 1)
                def _():
                    pltpu.make_async_copy(
                        sbufs[p], out_slice(b_start), osems[p]
                    ).wait()

        stage1()

        @pl.when(wid == _NW - 1)
        def _():
            pltpu.sync_copy(tailS, sbufs[0].at[pl.ds(0, 16), :])
            pltpu.sync_copy(
                sbufs[0].at[pl.ds(0, 16), :],
                s_hbm.at[pl.ds(_NSLAB * 32, 16), :],
            )

        magic_v[0, pl.ds(0, 16)] = jnp.full((16,), _MAGIC, jnp.int32)
        plsc.subcore_barrier()

        @pl.when(sid == 0)
        def _():
            pltpu.sync_copy(
                magic_v, flag.at[pl.ds(pl.multiple_of(8 * cid, 8), 8), :]
            )

        def poll_body(_):
            pltpu.sync_copy(
                flag.at[pl.ds(pl.multiple_of(8 * (1 - cid), 8), 8), :], fbuf
            )
            got = fbuf[0, pl.ds(0, 16)]
            ok = jax.lax.reduce_and(got == _MAGIC, axes=(0,))
            return jnp.logical_not(ok)

        lax.while_loop(lambda nd: nd, poll_body, jnp.bool_(True))

        def unit_of(step):
            u = step * _NW + wid
            h = u // _CPH
            c0 = pl.multiple_of((u % _CPH) * _CHUNK, 128)
            return h, c0

        def prep_and_fire(step, p):
            h, c0 = unit_of(step)
            pltpu.sync_copy(
                idxT.at[
                    pl.ds(pl.multiple_of((h // 8) * 8, 8), 8), pl.ds(c0, _CHUNK)
                ],
                idxbufs[p],
            )
            hr = h % 8
            for k in range(_CHUNK // 16):
                iv = idxbufs[p][hr, pl.ds(16 * k, 16)]
                glists[p][k // 8, pl.ds(16 * (k % 8), 16)] = (
                    jax.lax.shift_right_logical(iv, 2)
                )
                mrows[p][pl.ds(16 * k, 16)] = jax.lax.shift_left(iv & 3, 5)
            for q in range(_CHUNK // 128):
                pltpu.async_copy(
                    s_hbm.at[glists[p].at[q]],
                    gbufs[p].at[pl.ds(128 * q, 128), :],
                    gsems[p],
                )

        def extract(step, p):
            gb, mr = gbufs[p], mrows[p]
            for half in range(2):
                ob = obufs[p][half]

                def c_body(c, _):
                    lbase = 16 * c

                    def d_body(dg, _):
                        outs = []
                        for dd in range(2):
                            d = 2 * dg + dd
                            lane = lbase + ((iota + d) & 15)
                            b_l = 128 * half + lane
                            mvp = plsc.load_gather(mr, [b_l])
                            for j0 in (0, 16):
                                v = plsc.load_gather(
                                    gb, [b_l, mvp + (iota + j0)]
                                )
                                outs.append((j0, lane, v))
                        for j0, lane, v in outs:
                            plsc.store_scatter(ob, [iota + j0, lane], v)
                        return 0

                    lax.fori_loop(0, 8, d_body, 0)
                    return 0

                lax.fori_loop(0, 8, c_body, 0)

        def write_out(step, p):
            h, c0 = unit_of(step)
            for half in range(2):
                pltpu.async_copy(
                    obufs[p][half],
                    outP.at[h, :, pl.ds(c0 + 128 * half, 128)],
                    wsems[p],
                )

        def wait_write(step, p):
            h, c0 = unit_of(step)
            for half in range(2):
                pltpu.make_async_copy(
                    obufs[p][half],
                    outP.at[h, :, pl.ds(c0 + 128 * half, 128)],
                    wsems[p],
                ).wait()

        def drain_gather(p):
            pltpu.make_async_copy(
                s_hbm.at[pl.ds(0, _CHUNK)], gbufs[p], gsems[p]
            ).wait()

        prep_and_fire(0, 0)

        def s2_group(g, _):
            for p in range(2):
                step = 2 * g + p

                @pl.when(step < _STEPS)
                def _():
                    drain_gather(p)

                    @pl.when(step + 1 < _STEPS)
                    def _():
                        prep_and_fire(step + 1, 1 - p)

                    @pl.when(step - 2 >= 0)
                    def _():
                        wait_write(step - 2, p)

                    extract(step, p)
                    write_out(step, p)

            return 0

        lax.fori_loop(0, (_STEPS + 1) // 2, s2_group, 0)
        for step in (_STEPS - 2, _STEPS - 1):
            wait_write(step, step % 2)

    return phys_kernel


def kernel(indices, table):
    idxT = indices.T.astype(jnp.int32)
    tabT = table.T
    tailS = table[_NSLAB * 128:].reshape(16, 128)
    outP, _, _ = _make_phys_kernel()(idxT, tabT, tailS)
    return outP.transpose(2, 0, 1)
